# baseline (speedup 1.0000x reference)
import functools

import jax
import jax.numpy as jnp
from jax import lax
from jax.experimental import pallas as pl
from jax.experimental.pallas import tpu as pltpu
from jax.experimental.pallas import tpu_sc as plsc

N = 10000
NP = 10240
E = 320000
F_IN = 128
DIM = 10
DP = 16
G = 64

NC = 2
NS = 16
NW = NC * NS
EPW = E // NW
CH = 80
NCHUNK = EPW // CH
RPS = NP // NS
ZBLK = 128

_mesh = plsc.VectorSubcoreMesh(core_axis_name="c", subcore_axis_name="s")
_sc_params = pltpu.CompilerParams(use_tc_tiling_on_sc=False)
_sc_params2 = pltpu.CompilerParams(use_tc_tiling_on_sc=False, needs_layout_passes=False)


NR = NP // 8


def _mm_body(x_ref, w_ref, o_ref):
    h = jax.lax.dot_general(
        x_ref[...], w_ref[...], (((1,), (0,)), ((), ())),
        preferred_element_type=jnp.float32,
        precision=jax.lax.Precision.HIGHEST,
    )
    o_ref[...] = jnp.concatenate(
        [h, jnp.zeros((h.shape[0], 128 - DP), jnp.float32)], axis=1)


def _project(x, w1pt):
    blk = N // 10
    return pl.pallas_call(
        _mm_body,
        grid=(10,),
        in_specs=[
            pl.BlockSpec((blk, F_IN), lambda i: (i, 0)),
            pl.BlockSpec((F_IN, DP), lambda i: (0, 0)),
        ],
        out_specs=pl.BlockSpec((blk, 128), lambda i: (i, 0)),
        out_shape=jax.ShapeDtypeStruct((NP, 128), jnp.float32),
    )(x, w1pt)


IB = 25
NBLK = NCHUNK // IB


@functools.partial(
    pl.kernel,
    mesh=_mesh,
    compiler_params=_sc_params,
    out_type=jax.ShapeDtypeStruct((NC, NP, DP), jnp.float32),
    scratch_types=[
        pltpu.VMEM((EPW,), jnp.int32),
        pltpu.VMEM((EPW,), jnp.int32),
        pltpu.VMEM((IB, CH, DP), jnp.float32),
        pltpu.VMEM((ZBLK, DP), jnp.float32),
        pltpu.VMEM_SHARED((NP, DP), jnp.float32),
        pltpu.VMEM_SHARED((NP, DP), jnp.float32),
        pltpu.SemaphoreType.DMA,
        pltpu.SemaphoreType.DMA,
        pltpu.SemaphoreType.DMA,
    ],
)
def _sc_pass1(h_hbm, edge_hbm, out_hbm, src_v, dst_v, rows3, zblk,
              h_sh, acc_sh, sem_g, sem_g2, sem_s):
    cid = lax.axis_index("c")
    sid = lax.axis_index("s")
    wid = sid * NC + cid
    e0 = wid * EPW

    stage = pltpu.async_copy(h_hbm.at[pl.ds(sid * RPS, RPS), pl.ds(0, DP)],
                             h_sh.at[pl.ds(sid * RPS, RPS)], sem_g)
    ld_s = pltpu.async_copy(edge_hbm.at[0, pl.ds(e0, EPW)], src_v, sem_g2)
    ld_d = pltpu.async_copy(edge_hbm.at[1, pl.ds(e0, EPW)], dst_v, sem_g2)

    @pl.loop(0, ZBLK)
    def _(i):
        zblk[i, :] = jnp.zeros((DP,), jnp.float32)

    @pl.loop(0, RPS // ZBLK)
    def _(j):
        pltpu.sync_copy(zblk, acc_sh.at[pl.ds(sid * RPS + j * ZBLK, ZBLK)])

    stage.wait()
    ld_s.wait()
    ld_d.wait()
    plsc.subcore_barrier()

    HB = IB // 2

    @pl.loop(0, NBLK)
    def _(b):
        base = b * IB * CH

        def idx(j):
            return src_v.at[pl.ds(base + j * CH, CH)]

        def didx(j):
            return dst_v.at[pl.ds(base + j * CH, CH)]

        g1 = [pltpu.async_copy(h_sh.at[idx(j)], rows3.at[j], sem_g)
              for j in range(HB)]
        g2 = [pltpu.async_copy(h_sh.at[idx(j)], rows3.at[j], sem_g2)
              for j in range(HB, IB)]
        for g in g1:
            g.wait()
        s1 = [pltpu.async_copy(rows3.at[j], acc_sh.at[didx(j)], sem_s, add=True)
              for j in range(HB)]
        for g in g2:
            g.wait()
        s2 = [pltpu.async_copy(rows3.at[j], acc_sh.at[didx(j)], sem_s, add=True)
              for j in range(HB, IB)]
        for s in s1 + s2:
            s.wait()

    plsc.subcore_barrier()
    pltpu.sync_copy(acc_sh.at[pl.ds(sid * RPS, RPS)],
                    out_hbm.at[cid, pl.ds(sid * RPS, RPS)])


@functools.partial(
    pl.kernel,
    mesh=_mesh,
    compiler_params=_sc_params2,
    out_type=jax.ShapeDtypeStruct((NW // 2, 2 * G), jnp.float32),
    scratch_types=[
        pltpu.VMEM((RPS, DP), jnp.float32),
        pltpu.VMEM((RPS, DP), jnp.float32),
        pltpu.VMEM((DIM, 16), jnp.float32),
        pltpu.VMEM((RPS,), jnp.float32),
        pltpu.VMEM((NP,), jnp.float32),
        pltpu.VMEM((N,), jnp.int32),
        pltpu.VMEM((EPW,), jnp.int32),
        pltpu.VMEM((EPW,), jnp.int32),
        pltpu.VMEM((G * 16,), jnp.float32),
        pltpu.VMEM((G,), jnp.float32),
        pltpu.VMEM_SHARED((NP,), jnp.float32),
        pltpu.SemaphoreType.DMA,
    ],
)
def _sc_pass2(parts_hbm, batch_hbm, edge_hbm, vb_hbm, out_hbm,
              p0_v, p1_v, vb_v, zbuf, z_v, batch_v, src_v, dst_v, tacc, tred,
              z_sh, sem):
    cid = lax.axis_index("c")
    sid = lax.axis_index("s")
    wid = sid * NC + cid
    e0 = wid * EPW
    r0 = sid * RPS

    zcps = [
        pltpu.async_copy(parts_hbm.at[0, pl.ds(r0, RPS)], p0_v, sem),
        pltpu.async_copy(parts_hbm.at[1, pl.ds(r0, RPS)], p1_v, sem),
        pltpu.async_copy(vb_hbm, vb_v, sem),
    ]
    cps = [
        pltpu.async_copy(batch_hbm, batch_v, sem),
        pltpu.async_copy(edge_hbm.at[0, pl.ds(e0, EPW)], src_v, sem),
        pltpu.async_copy(edge_hbm.at[1, pl.ds(e0, EPW)], dst_v, sem),
    ]

    @pl.loop(0, G)
    def _(i):
        tacc[pl.ds(i * 16, 16)] = jnp.zeros((16,), jnp.float32)

    for c in zcps:
        c.wait()

    lanes = lax.iota(jnp.int32, 16)

    @pl.loop(0, RPS // 16)
    def _(grp):
        rows = lanes + grp * 16
        zacc = jnp.zeros((16,), jnp.float32)
        for d in range(DIM):
            dcol = jnp.full((16,), d, jnp.int32)
            c0 = plsc.load_gather(p0_v, [rows, dcol])
            c1 = plsc.load_gather(p1_v, [rows, dcol])
            col = jnp.maximum(c0 + c1, 0.0)
            zacc = zacc + col * vb_v[d, :]
        zbuf[pl.ds(grp * 16, 16)] = zacc

    pltpu.sync_copy(zbuf, z_sh.at[pl.ds(r0, RPS)])
    plsc.subcore_barrier()
    pltpu.sync_copy(z_sh, z_v)

    for c in cps:
        c.wait()

    @pl.loop(0, EPW // 64)
    def _(i):
        for j in range(4):
            s16 = src_v[pl.ds(i * 64 + j * 16, 16)]
            d16 = dst_v[pl.ds(i * 64 + j * 16, 16)]
            zv = plsc.load_gather(z_v, [s16])
            gv = plsc.load_gather(batch_v, [d16])
            plsc.addupdate_scatter(tacc, [gv * 16 + lanes], zv)

    @pl.loop(0, G // 16)
    def _(k):
        acc = jnp.zeros((16,), jnp.float32)
        gbase = (k * 16 + lanes) * 16
        for lane in range(16):
            acc = acc + plsc.load_gather(tacc, [gbase + lane])
        tred[pl.ds(k * 16, 16)] = acc

    pltpu.sync_copy(tred, out_hbm.at[wid // 2, pl.ds((wid % 2) * G, G)])


def _fin_body(t_ref, b_ref, o_ref):
    m = t_ref[...]
    t = jnp.sum(m[:, :G] + m[:, G:], axis=0)
    b = b_ref[...]
    gi = lax.broadcasted_iota(jnp.int32, (G, b.shape[1]), 0)
    cnt = jnp.sum((b == gi).astype(jnp.float32), axis=1)
    o_ref[...] = jax.nn.sigmoid(t / jnp.maximum(cnt, 1.0))[:, None]


def _finish(tparts, batch_padded):
    return pl.pallas_call(
        _fin_body,
        in_specs=[
            pl.BlockSpec(tparts.shape, lambda: (0, 0)),
            pl.BlockSpec(batch_padded.shape, lambda: (0, 0)),
        ],
        out_specs=pl.BlockSpec((G, 1), lambda: (0, 0)),
        out_shape=jax.ShapeDtypeStruct((G, 1), jnp.float32),
    )(tparts, batch_padded)


def kernel(x, edge_index, batch, W1, W2, Wfc):
    w1pt = jnp.zeros((F_IN, DP), jnp.float32).at[:, :DIM].set(W1.T)
    v = W2.T @ Wfc.T
    vb = jnp.broadcast_to(v, (DIM, 16)).astype(jnp.float32)

    h = _project(x, w1pt)
    parts = _sc_pass1(h, edge_index)
    tparts = _sc_pass2(parts, batch, edge_index, vb)

    npad = 10240
    batch_padded = jnp.full((1, npad), G, jnp.int32).at[0, :N].set(batch)
    return _finish(tparts, batch_padded)


# speedup vs baseline: 1.0166x; 1.0166x over previous
import functools

import jax
import jax.numpy as jnp
from jax import lax
from jax.experimental import pallas as pl
from jax.experimental.pallas import tpu as pltpu
from jax.experimental.pallas import tpu_sc as plsc

N = 10000
NP = 10240
E = 320000
F_IN = 128
DIM = 10
DP = 16
G = 64

NC = 2
NS = 16
NW = NC * NS
EPW = E // NW
CH = 80
NCHUNK = EPW // CH
RPS = NP // NS
ZBLK = 128

_mesh = plsc.VectorSubcoreMesh(core_axis_name="c", subcore_axis_name="s")
_sc_params = pltpu.CompilerParams(use_tc_tiling_on_sc=False)
_sc_params2 = pltpu.CompilerParams(use_tc_tiling_on_sc=False, needs_layout_passes=False)


NR = NP // 8


def _mm_body(x_ref, w_ref, o_ref):
    h = jax.lax.dot_general(
        x_ref[...], w_ref[...], (((1,), (0,)), ((), ())),
        preferred_element_type=jnp.float32,
        precision=jax.lax.Precision.HIGHEST,
    )
    o_ref[...] = jnp.concatenate(
        [h, jnp.zeros((h.shape[0], 128 - DP), jnp.float32)], axis=1)


def _project(x, w1pt):
    blk = N // 2
    return pl.pallas_call(
        _mm_body,
        grid=(2,),
        in_specs=[
            pl.BlockSpec((blk, F_IN), lambda i: (i, 0)),
            pl.BlockSpec((F_IN, DP), lambda i: (0, 0)),
        ],
        out_specs=pl.BlockSpec((blk, 128), lambda i: (i, 0)),
        out_shape=jax.ShapeDtypeStruct((NP, 128), jnp.float32),
    )(x, w1pt)


IB = 25
NBLK = NCHUNK // IB


@functools.partial(
    pl.kernel,
    mesh=_mesh,
    compiler_params=_sc_params,
    out_type=jax.ShapeDtypeStruct((NC, NP, DP), jnp.float32),
    scratch_types=[
        pltpu.VMEM((EPW,), jnp.int32),
        pltpu.VMEM((EPW,), jnp.int32),
        pltpu.VMEM((IB, CH, DP), jnp.float32),
        pltpu.VMEM((ZBLK, DP), jnp.float32),
        pltpu.VMEM_SHARED((NP, DP), jnp.float32),
        pltpu.VMEM_SHARED((NP, DP), jnp.float32),
        pltpu.SemaphoreType.DMA,
        pltpu.SemaphoreType.DMA,
        pltpu.SemaphoreType.DMA,
    ],
)
def _sc_pass1(h_hbm, edge_hbm, out_hbm, src_v, dst_v, rows3, zblk,
              h_sh, acc_sh, sem_g, sem_g2, sem_s):
    cid = lax.axis_index("c")
    sid = lax.axis_index("s")
    wid = sid * NC + cid
    e0 = wid * EPW

    stage = pltpu.async_copy(h_hbm.at[pl.ds(sid * RPS, RPS), pl.ds(0, DP)],
                             h_sh.at[pl.ds(sid * RPS, RPS)], sem_g)
    ld_s = pltpu.async_copy(edge_hbm.at[0, pl.ds(e0, EPW)], src_v, sem_g2)
    ld_d = pltpu.async_copy(edge_hbm.at[1, pl.ds(e0, EPW)], dst_v, sem_g2)

    @pl.loop(0, ZBLK)
    def _(i):
        zblk[i, :] = jnp.zeros((DP,), jnp.float32)

    @pl.loop(0, RPS // ZBLK)
    def _(j):
        pltpu.sync_copy(zblk, acc_sh.at[pl.ds(sid * RPS + j * ZBLK, ZBLK)])

    stage.wait()
    ld_s.wait()
    ld_d.wait()
    plsc.subcore_barrier()

    HB = IB // 2

    @pl.loop(0, NBLK)
    def _(b):
        base = b * IB * CH

        def idx(j):
            return src_v.at[pl.ds(base + j * CH, CH)]

        def didx(j):
            return dst_v.at[pl.ds(base + j * CH, CH)]

        g1 = [pltpu.async_copy(h_sh.at[idx(j)], rows3.at[j], sem_g)
              for j in range(HB)]
        g2 = [pltpu.async_copy(h_sh.at[idx(j)], rows3.at[j], sem_g2)
              for j in range(HB, IB)]
        for g in g1:
            g.wait()
        s1 = [pltpu.async_copy(rows3.at[j], acc_sh.at[didx(j)], sem_s, add=True)
              for j in range(HB)]
        for g in g2:
            g.wait()
        s2 = [pltpu.async_copy(rows3.at[j], acc_sh.at[didx(j)], sem_s, add=True)
              for j in range(HB, IB)]
        for s in s1 + s2:
            s.wait()

    plsc.subcore_barrier()
    pltpu.sync_copy(acc_sh.at[pl.ds(sid * RPS, RPS)],
                    out_hbm.at[cid, pl.ds(sid * RPS, RPS)])


@functools.partial(
    pl.kernel,
    mesh=_mesh,
    compiler_params=_sc_params2,
    out_type=jax.ShapeDtypeStruct((NW // 2, 2 * G), jnp.float32),
    scratch_types=[
        pltpu.VMEM((RPS, DP), jnp.float32),
        pltpu.VMEM((RPS, DP), jnp.float32),
        pltpu.VMEM((DIM, 16), jnp.float32),
        pltpu.VMEM((RPS,), jnp.float32),
        pltpu.VMEM((NP,), jnp.float32),
        pltpu.VMEM((N,), jnp.int32),
        pltpu.VMEM((EPW,), jnp.int32),
        pltpu.VMEM((EPW,), jnp.int32),
        pltpu.VMEM((G * 16,), jnp.float32),
        pltpu.VMEM((G,), jnp.float32),
        pltpu.VMEM_SHARED((NP,), jnp.float32),
        pltpu.SemaphoreType.DMA,
    ],
)
def _sc_pass2(parts_hbm, batch_hbm, edge_hbm, vb_hbm, out_hbm,
              p0_v, p1_v, vb_v, zbuf, z_v, batch_v, src_v, dst_v, tacc, tred,
              z_sh, sem):
    cid = lax.axis_index("c")
    sid = lax.axis_index("s")
    wid = sid * NC + cid
    e0 = wid * EPW
    r0 = sid * RPS

    zcps = [
        pltpu.async_copy(parts_hbm.at[0, pl.ds(r0, RPS)], p0_v, sem),
        pltpu.async_copy(parts_hbm.at[1, pl.ds(r0, RPS)], p1_v, sem),
        pltpu.async_copy(vb_hbm, vb_v, sem),
    ]
    cps = [
        pltpu.async_copy(batch_hbm, batch_v, sem),
        pltpu.async_copy(edge_hbm.at[0, pl.ds(e0, EPW)], src_v, sem),
        pltpu.async_copy(edge_hbm.at[1, pl.ds(e0, EPW)], dst_v, sem),
    ]

    @pl.loop(0, G)
    def _(i):
        tacc[pl.ds(i * 16, 16)] = jnp.zeros((16,), jnp.float32)

    for c in zcps:
        c.wait()

    lanes = lax.iota(jnp.int32, 16)

    @pl.loop(0, RPS // 16)
    def _(grp):
        rows = lanes + grp * 16
        zacc = jnp.zeros((16,), jnp.float32)
        for d in range(DIM):
            dcol = jnp.full((16,), d, jnp.int32)
            c0 = plsc.load_gather(p0_v, [rows, dcol])
            c1 = plsc.load_gather(p1_v, [rows, dcol])
            col = jnp.maximum(c0 + c1, 0.0)
            zacc = zacc + col * vb_v[d, :]
        zbuf[pl.ds(grp * 16, 16)] = zacc

    pltpu.sync_copy(zbuf, z_sh.at[pl.ds(r0, RPS)])
    plsc.subcore_barrier()
    pltpu.sync_copy(z_sh, z_v)

    for c in cps:
        c.wait()

    @pl.loop(0, EPW // 64)
    def _(i):
        for j in range(4):
            s16 = src_v[pl.ds(i * 64 + j * 16, 16)]
            d16 = dst_v[pl.ds(i * 64 + j * 16, 16)]
            zv = plsc.load_gather(z_v, [s16])
            gv = plsc.load_gather(batch_v, [d16])
            plsc.addupdate_scatter(tacc, [gv * 16 + lanes], zv)

    @pl.loop(0, G // 16)
    def _(k):
        acc = jnp.zeros((16,), jnp.float32)
        gbase = (k * 16 + lanes) * 16
        for lane in range(16):
            acc = acc + plsc.load_gather(tacc, [gbase + lane])
        tred[pl.ds(k * 16, 16)] = acc

    pltpu.sync_copy(tred, out_hbm.at[wid // 2, pl.ds((wid % 2) * G, G)])


def _fin_body(t_ref, b_ref, o_ref):
    m = t_ref[...]
    t = jnp.sum(m[:, :G] + m[:, G:], axis=0)
    b = b_ref[...]
    gi = lax.broadcasted_iota(jnp.int32, (G, b.shape[1]), 0)
    cnt = jnp.sum((b == gi).astype(jnp.float32), axis=1)
    o_ref[...] = jax.nn.sigmoid(t / jnp.maximum(cnt, 1.0))[:, None]


def _finish(tparts, batch_padded):
    return pl.pallas_call(
        _fin_body,
        in_specs=[
            pl.BlockSpec(tparts.shape, lambda: (0, 0)),
            pl.BlockSpec(batch_padded.shape, lambda: (0, 0)),
        ],
        out_specs=pl.BlockSpec((G, 1), lambda: (0, 0)),
        out_shape=jax.ShapeDtypeStruct((G, 1), jnp.float32),
    )(tparts, batch_padded)


def kernel(x, edge_index, batch, W1, W2, Wfc):
    w1pt = jnp.zeros((F_IN, DP), jnp.float32).at[:, :DIM].set(W1.T)
    v = W2.T @ Wfc.T
    vb = jnp.broadcast_to(v, (DIM, 16)).astype(jnp.float32)

    h = _project(x, w1pt)
    parts = _sc_pass1(h, edge_index)
    tparts = _sc_pass2(parts, batch, edge_index, vb)

    npad = 10240
    batch_padded = jnp.full((1, npad), G, jnp.int32).at[0, :N].set(batch)
    return _finish(tparts, batch_padded)


# speedup vs baseline: 1.0597x; 1.0424x over previous
import functools

import jax
import jax.numpy as jnp
from jax import lax
from jax.experimental import pallas as pl
from jax.experimental.pallas import tpu as pltpu
from jax.experimental.pallas import tpu_sc as plsc

N = 10000
NP = 10240
E = 320000
F_IN = 128
DIM = 10
DP = 16
G = 64

NC = 2
NS = 16
NW = NC * NS
EPW = E // NW
CH = 80
NCHUNK = EPW // CH
RPS = NP // NS
ZBLK = 128

_mesh = plsc.VectorSubcoreMesh(core_axis_name="c", subcore_axis_name="s")
_sc_params = pltpu.CompilerParams(use_tc_tiling_on_sc=False)
_sc_params2 = pltpu.CompilerParams(use_tc_tiling_on_sc=False, needs_layout_passes=False)


NR = NP // 8


def _mm_body(x_ref, w_ref, o_ref):
    h = jax.lax.dot_general(
        x_ref[...], w_ref[...], (((1,), (0,)), ((), ())),
        preferred_element_type=jnp.float32,
        precision=jax.lax.Precision.HIGHEST,
    )
    o_ref[...] = jnp.concatenate(
        [h, jnp.zeros((h.shape[0], 128 - DP), jnp.float32)], axis=1)


def _project(x, w1pt):
    blk = N // 5
    return pl.pallas_call(
        _mm_body,
        grid=(5,),
        in_specs=[
            pl.BlockSpec((blk, F_IN), lambda i: (i, 0)),
            pl.BlockSpec((F_IN, DP), lambda i: (0, 0)),
        ],
        out_specs=pl.BlockSpec((blk, 128), lambda i: (i, 0)),
        out_shape=jax.ShapeDtypeStruct((NP, 128), jnp.float32),
    )(x, w1pt)


IB = 25
NBLK = NCHUNK // IB


@functools.partial(
    pl.kernel,
    mesh=_mesh,
    compiler_params=_sc_params,
    out_type=jax.ShapeDtypeStruct((NC, NP, DP), jnp.float32),
    scratch_types=[
        pltpu.VMEM((EPW,), jnp.int32),
        pltpu.VMEM((EPW,), jnp.int32),
        pltpu.VMEM((IB, CH, DP), jnp.float32),
        pltpu.VMEM((ZBLK, DP), jnp.float32),
        pltpu.VMEM_SHARED((NP, DP), jnp.float32),
        pltpu.VMEM_SHARED((NP, DP), jnp.float32),
        pltpu.SemaphoreType.DMA,
        pltpu.SemaphoreType.DMA,
        pltpu.SemaphoreType.DMA,
    ],
)
def _sc_pass1(h_hbm, edge_hbm, out_hbm, src_v, dst_v, rows3, zblk,
              h_sh, acc_sh, sem_g, sem_g2, sem_s):
    cid = lax.axis_index("c")
    sid = lax.axis_index("s")
    wid = sid * NC + cid
    e0 = wid * EPW

    stage = pltpu.async_copy(h_hbm.at[pl.ds(sid * RPS, RPS), pl.ds(0, DP)],
                             h_sh.at[pl.ds(sid * RPS, RPS)], sem_g)
    ld_s = pltpu.async_copy(edge_hbm.at[0, pl.ds(e0, EPW)], src_v, sem_g2)
    ld_d = pltpu.async_copy(edge_hbm.at[1, pl.ds(e0, EPW)], dst_v, sem_g2)

    @pl.loop(0, ZBLK)
    def _(i):
        zblk[i, :] = jnp.zeros((DP,), jnp.float32)

    @pl.loop(0, RPS // ZBLK)
    def _(j):
        pltpu.sync_copy(zblk, acc_sh.at[pl.ds(sid * RPS + j * ZBLK, ZBLK)])

    stage.wait()
    ld_s.wait()
    ld_d.wait()
    plsc.subcore_barrier()

    HB = IB // 2

    @pl.loop(0, NBLK)
    def _(b):
        base = b * IB * CH

        def idx(j):
            return src_v.at[pl.ds(base + j * CH, CH)]

        def didx(j):
            return dst_v.at[pl.ds(base + j * CH, CH)]

        g1 = [pltpu.async_copy(h_sh.at[idx(j)], rows3.at[j], sem_g)
              for j in range(HB)]
        g2 = [pltpu.async_copy(h_sh.at[idx(j)], rows3.at[j], sem_g2)
              for j in range(HB, IB)]
        for g in g1:
            g.wait()
        s1 = [pltpu.async_copy(rows3.at[j], acc_sh.at[didx(j)], sem_s, add=True)
              for j in range(HB)]
        for g in g2:
            g.wait()
        s2 = [pltpu.async_copy(rows3.at[j], acc_sh.at[didx(j)], sem_s, add=True)
              for j in range(HB, IB)]
        for s in s1 + s2:
            s.wait()

    plsc.subcore_barrier()
    pltpu.sync_copy(acc_sh.at[pl.ds(sid * RPS, RPS)],
                    out_hbm.at[cid, pl.ds(sid * RPS, RPS)])


@functools.partial(
    pl.kernel,
    mesh=_mesh,
    compiler_params=_sc_params2,
    out_type=jax.ShapeDtypeStruct((NW // 2, 2 * G), jnp.float32),
    scratch_types=[
        pltpu.VMEM((RPS, DP), jnp.float32),
        pltpu.VMEM((RPS, DP), jnp.float32),
        pltpu.VMEM((DIM, 16), jnp.float32),
        pltpu.VMEM((RPS,), jnp.float32),
        pltpu.VMEM((NP,), jnp.float32),
        pltpu.VMEM((N,), jnp.int32),
        pltpu.VMEM((EPW,), jnp.int32),
        pltpu.VMEM((EPW,), jnp.int32),
        pltpu.VMEM((G * 16,), jnp.float32),
        pltpu.VMEM((G,), jnp.float32),
        pltpu.VMEM_SHARED((NP,), jnp.float32),
        pltpu.SemaphoreType.DMA,
    ],
)
def _sc_pass2(parts_hbm, batch_hbm, edge_hbm, vb_hbm, out_hbm,
              p0_v, p1_v, vb_v, zbuf, z_v, batch_v, src_v, dst_v, tacc, tred,
              z_sh, sem):
    cid = lax.axis_index("c")
    sid = lax.axis_index("s")
    wid = sid * NC + cid
    e0 = wid * EPW
    r0 = sid * RPS

    zcps = [
        pltpu.async_copy(parts_hbm.at[0, pl.ds(r0, RPS)], p0_v, sem),
        pltpu.async_copy(parts_hbm.at[1, pl.ds(r0, RPS)], p1_v, sem),
        pltpu.async_copy(vb_hbm, vb_v, sem),
    ]
    cps = [
        pltpu.async_copy(batch_hbm, batch_v, sem),
        pltpu.async_copy(edge_hbm.at[0, pl.ds(e0, EPW)], src_v, sem),
        pltpu.async_copy(edge_hbm.at[1, pl.ds(e0, EPW)], dst_v, sem),
    ]

    @pl.loop(0, G)
    def _(i):
        tacc[pl.ds(i * 16, 16)] = jnp.zeros((16,), jnp.float32)

    for c in zcps:
        c.wait()

    lanes = lax.iota(jnp.int32, 16)

    @pl.loop(0, RPS // 16)
    def _(grp):
        rows = lanes + grp * 16
        zacc = jnp.zeros((16,), jnp.float32)
        for d in range(DIM):
            dcol = jnp.full((16,), d, jnp.int32)
            c0 = plsc.load_gather(p0_v, [rows, dcol])
            c1 = plsc.load_gather(p1_v, [rows, dcol])
            col = jnp.maximum(c0 + c1, 0.0)
            zacc = zacc + col * vb_v[d, :]
        zbuf[pl.ds(grp * 16, 16)] = zacc

    pltpu.sync_copy(zbuf, z_sh.at[pl.ds(r0, RPS)])
    plsc.subcore_barrier()
    pltpu.sync_copy(z_sh, z_v)

    for c in cps:
        c.wait()

    @pl.loop(0, EPW // 64)
    def _(i):
        for j in range(4):
            s16 = src_v[pl.ds(i * 64 + j * 16, 16)]
            d16 = dst_v[pl.ds(i * 64 + j * 16, 16)]
            zv = plsc.load_gather(z_v, [s16])
            gv = plsc.load_gather(batch_v, [d16])
            plsc.addupdate_scatter(tacc, [gv * 16 + lanes], zv)

    @pl.loop(0, G // 16)
    def _(k):
        acc = jnp.zeros((16,), jnp.float32)
        gbase = (k * 16 + lanes) * 16
        for lane in range(16):
            acc = acc + plsc.load_gather(tacc, [gbase + lane])
        tred[pl.ds(k * 16, 16)] = acc

    pltpu.sync_copy(tred, out_hbm.at[wid // 2, pl.ds((wid % 2) * G, G)])


def _fin_body(t_ref, b_ref, o_ref):
    m = t_ref[...]
    t = jnp.sum(m[:, :G] + m[:, G:], axis=0)
    b = b_ref[...]
    gi = lax.broadcasted_iota(jnp.int32, (G, b.shape[1]), 0)
    cnt = jnp.sum((b == gi).astype(jnp.float32), axis=1)
    o_ref[...] = jax.nn.sigmoid(t / jnp.maximum(cnt, 1.0))[:, None]


def _finish(tparts, batch_padded):
    return pl.pallas_call(
        _fin_body,
        in_specs=[
            pl.BlockSpec(tparts.shape, lambda: (0, 0)),
            pl.BlockSpec(batch_padded.shape, lambda: (0, 0)),
        ],
        out_specs=pl.BlockSpec((G, 1), lambda: (0, 0)),
        out_shape=jax.ShapeDtypeStruct((G, 1), jnp.float32),
    )(tparts, batch_padded)


def kernel(x, edge_index, batch, W1, W2, Wfc):
    w1pt = jnp.zeros((F_IN, DP), jnp.float32).at[:, :DIM].set(W1.T)
    v = W2.T @ Wfc.T
    vb = jnp.broadcast_to(v, (DIM, 16)).astype(jnp.float32)

    h = _project(x, w1pt)
    parts = _sc_pass1(h, edge_index)
    tparts = _sc_pass2(parts, batch, edge_index, vb)

    npad = 10240
    batch_padded = jnp.full((1, npad), G, jnp.int32).at[0, :N].set(batch)
    return _finish(tparts, batch_padded)
